# baseline (speedup 1.0000x reference)
import jax
import jax.numpy as jnp
from jax.experimental import pallas as pl
from jax.experimental.pallas import tpu as pltpu

_K = 16
_S = 2


def _bdot(a, b):
    return jnp.dot(a.astype(jnp.bfloat16), b.astype(jnp.bfloat16),
                   preferred_element_type=jnp.float32)


def _pt_layer_kernel(x_ref, wq_ref, wk_ref, wv_ref, wg_ref, wo_ref,
                     bq_ref, bk_ref, bv_ref, bg_ref, bo_ref, out_ref,
                     xv_ref, yv_ref, isem, osem):
    B = x_ref.shape[0]
    N = x_ref.shape[2]
    tn = N // _S
    C = wq_ref.shape[0]

    def in_copy(b, s):
        return pltpu.make_async_copy(
            x_ref.at[b, :, pl.ds(s * tn, tn)],
            xv_ref.at[b, :, pl.ds(s * tn, tn)],
            isem.at[b * _S + s])

    def out_copy(b, s):
        return pltpu.make_async_copy(
            yv_ref.at[b, :, pl.ds(s * tn, tn)],
            out_ref.at[b, :, pl.ds(s * tn, tn)],
            osem.at[b * _S + s])

    for b in range(B):
        for s in range(_S):
            in_copy(b, s).start()

    ii = jax.lax.broadcasted_iota(jnp.int32, (C, C), 0)
    jj = jax.lax.broadcasted_iota(jnp.int32, (C, C), 1)
    eye = (ii == jj).astype(jnp.float32)

    def _col(b_ref):
        return jnp.sum(eye * b_ref[...][None, :], axis=1, keepdims=True)

    wqk = (wq_ref[...] - wk_ref[...]).astype(jnp.bfloat16)
    bqk = _col(bq_ref) - _col(bk_ref)
    wo = wo_ref[...]
    wog = wo + jnp.dot(wo, wg_ref[...], preferred_element_type=jnp.float32)
    wog = wog.astype(jnp.bfloat16)
    bog = jnp.dot(wo, _col(bg_ref), preferred_element_type=jnp.float32)
    bog = bog + _col(bo_ref)
    wv = wv_ref[...].astype(jnp.bfloat16)
    bv = _col(bv_ref)

    for b in range(B):
        for s in range(_S):
            in_copy(b, s).wait()
        xb = xv_ref[b].astype(jnp.bfloat16)
        sc = _bdot(wqk, xb) + bqk
        m = jnp.max(sc, axis=0, keepdims=True)
        e = jnp.exp(sc - m)
        attn = e / jnp.sum(e, axis=0, keepdims=True)
        v = _bdot(wv, xb) + bv
        xa = (float(_K) * attn) * v
        yv_ref[b] = _bdot(wog, xa) + bog
        for s in range(_S):
            out_copy(b, s).start()

    for b in range(B):
        for s in range(_S):
            out_copy(b, s).wait()


@jax.jit
def kernel(x, pos, Wq, bq, Wk, bk, Wv, bv, Wg, bg, Wo, bo):
    del pos
    B, C_in, N = x.shape
    C_out = Wq.shape[0]

    wspec = pl.BlockSpec((C_out, C_in), lambda: (0, 0))
    bspec = pl.BlockSpec((C_out,), lambda: (0,))
    anyspec = pl.BlockSpec(memory_space=pl.MemorySpace.ANY)

    out = pl.pallas_call(
        _pt_layer_kernel,
        grid=(),
        in_specs=[
            anyspec,
            wspec, wspec, wspec, wspec, wspec,
            bspec, bspec, bspec, bspec, bspec,
        ],
        out_specs=anyspec,
        out_shape=jax.ShapeDtypeStruct((B, C_out, N), jnp.float32),
        scratch_shapes=[
            pltpu.VMEM((B, C_in, N), jnp.float32),
            pltpu.VMEM((B, C_out, N), jnp.float32),
            pltpu.SemaphoreType.DMA((B * _S,)),
            pltpu.SemaphoreType.DMA((B * _S,)),
        ],
    )(x, Wq, Wk, Wv, Wg, Wo, bq, bk, bv, bg, bo)
    return out
